# baseline (speedup 1.0000x reference)
import functools

import jax
import jax.numpy as jnp
from jax import lax
from jax.experimental import pallas as pl
from jax.experimental.pallas import tpu as pltpu
from jax.experimental.pallas import tpu_sc as plsc

D_MODEL = 1024
NUM_EXPERTS = 8
N_TOKENS = 4096
TOKEN_BLOCK = 1024
_LANES = 16
_TILES = 32
_TOK_PER_TILE = N_TOKENS // _TILES


def _logits_body(x_ref, gw_ref, gb_ref, out_ref):
    logits = (
        jnp.dot(x_ref[...], gw_ref[...], preferred_element_type=jnp.float32)
        + gb_ref[...]
    )
    out_ref[...] = logits.T


def _route_body(lg_ref, cm_ref, lt_ref, buf_ref, sem):
    wid = lax.axis_index("c") * 16 + lax.axis_index("s")
    base = wid * _TOK_PER_TILE
    for e in range(NUM_EXPERTS):
        pltpu.sync_copy(
            lg_ref.at[pl.ds(e * N_TOKENS + base, _TOK_PER_TILE)],
            lt_ref.at[pl.ds(e * _TOK_PER_TILE, _TOK_PER_TILE)],
        )
    for g in range(_TOK_PER_TILE // _LANES):
        le = [
            lt_ref[pl.ds(e * _TOK_PER_TILE + g * _LANES, _LANES)]
            for e in range(NUM_EXPERTS)
        ]
        m1 = le[0]
        for e in range(1, NUM_EXPERTS):
            m1 = jnp.maximum(m1, le[e])
        i1 = jnp.full((_LANES,), NUM_EXPERTS, jnp.int32)
        for e in range(NUM_EXPERTS - 1, -1, -1):
            i1 = jnp.where(le[e] == m1, e, i1)
        neg = jnp.full((_LANES,), -jnp.inf, jnp.float32)
        m2 = jnp.where(i1 == 0, neg, le[0])
        for e in range(1, NUM_EXPERTS):
            m2 = jnp.maximum(m2, jnp.where(i1 == e, neg, le[e]))
        i2 = jnp.full((_LANES,), NUM_EXPERTS, jnp.int32)
        for e in range(NUM_EXPERTS - 1, -1, -1):
            i2 = jnp.where((le[e] == m2) & (i1 != e), e, i2)
        c1 = 1.0 / (1.0 + jnp.exp(m2 - m1))
        c2 = 1.0 - c1
        zero = jnp.zeros((_LANES,), jnp.float32)
        for e in range(NUM_EXPERTS):
            colv = jnp.where(i1 == e, c1, zero) + jnp.where(i2 == e, c2, zero)
            buf_ref[pl.ds(e * _TOK_PER_TILE + g * _LANES, _LANES)] = colv
    for e in range(NUM_EXPERTS):
        pltpu.sync_copy(
            buf_ref.at[pl.ds(e * _TOK_PER_TILE, _TOK_PER_TILE)],
            cm_ref.at[pl.ds(e * N_TOKENS + base, _TOK_PER_TILE)],
        )


def _moe_body(x_ref, cmt_ref, ew_ref, eb_ref, out_ref):
    e = pl.program_id(1)
    cmt = cmt_ref[...]
    r2 = jax.lax.broadcasted_iota(jnp.int32, (NUM_EXPERTS, 2), 0)
    k2 = jax.lax.broadcasted_iota(jnp.int32, (NUM_EXPERTS, 2), 1)
    sel = (r2 == 2 * e + k2).astype(jnp.float32)
    cab = jax.lax.dot_general(
        cmt, sel, (((0,), (0,)), ((), ())), preferred_element_type=jnp.float32
    )
    ca = cab[:, 0:1]
    cb = cab[:, 1:2]
    ya = jnp.dot(x_ref[...], ew_ref[0], preferred_element_type=jnp.float32)
    yb = jnp.dot(x_ref[...], ew_ref[1], preferred_element_type=jnp.float32)
    y = ca * ya + cb * yb

    @pl.when(e == 0)
    def _init():
        out_ref[...] = y + jax.lax.dot_general(
            cmt,
            eb_ref[...],
            (((0,), (0,)), ((), ())),
            preferred_element_type=jnp.float32,
        )

    @pl.when(e != 0)
    def _acc():
        out_ref[...] = out_ref[...] + y


@jax.jit
def kernel(x, gate_W, gate_b, expert_W, expert_b):
    gb2 = gate_b.reshape(1, NUM_EXPERTS)
    logits_t = pl.pallas_call(
        _logits_body,
        grid=(1,),
        in_specs=[
            pl.BlockSpec((N_TOKENS, D_MODEL), lambda i: (0, 0)),
            pl.BlockSpec((D_MODEL, NUM_EXPERTS), lambda i: (0, 0)),
            pl.BlockSpec((1, NUM_EXPERTS), lambda i: (0, 0)),
        ],
        out_specs=pl.BlockSpec((NUM_EXPERTS, N_TOKENS), lambda i: (0, 0)),
        out_shape=jax.ShapeDtypeStruct((NUM_EXPERTS, N_TOKENS), jnp.float32),
    )(x, gate_W, gb2)

    mesh = plsc.VectorSubcoreMesh(core_axis_name="c", subcore_axis_name="s")
    route = functools.partial(
        pl.kernel,
        out_type=jax.ShapeDtypeStruct((N_TOKENS * NUM_EXPERTS,), jnp.float32),
        mesh=mesh,
        scratch_types=[
            pltpu.VMEM((NUM_EXPERTS * _TOK_PER_TILE,), jnp.float32),
            pltpu.VMEM((_TOK_PER_TILE * NUM_EXPERTS,), jnp.float32),
            pltpu.SemaphoreType.DMA,
        ],
    )(_route_body)
    cmat_t = route(logits_t.reshape(-1)).reshape(NUM_EXPERTS, N_TOKENS)

    n_tb = N_TOKENS // TOKEN_BLOCK
    return pl.pallas_call(
        _moe_body,
        grid=(n_tb, NUM_EXPERTS // 2),
        in_specs=[
            pl.BlockSpec((TOKEN_BLOCK, D_MODEL), lambda t, e: (t, 0)),
            pl.BlockSpec((NUM_EXPERTS, TOKEN_BLOCK), lambda t, e: (0, t)),
            pl.BlockSpec((2, D_MODEL, D_MODEL), lambda t, e: (e, 0, 0)),
            pl.BlockSpec((NUM_EXPERTS, D_MODEL), lambda t, e: (0, 0)),
        ],
        out_specs=pl.BlockSpec((TOKEN_BLOCK, D_MODEL), lambda t, e: (t, 0)),
        out_shape=jax.ShapeDtypeStruct((N_TOKENS, D_MODEL), jnp.float32),
    )(x, cmat_t, expert_W, expert_b)


# speedup vs baseline: 1.0642x; 1.0642x over previous
import functools

import jax
import jax.numpy as jnp
from jax import lax
from jax.experimental import pallas as pl
from jax.experimental.pallas import tpu as pltpu
from jax.experimental.pallas import tpu_sc as plsc

D_MODEL = 1024
NUM_EXPERTS = 8
N_TOKENS = 4096
TOKEN_BLOCK = 1024
_LANES = 16
_TILES = 16
_TOK_PER_TILE = N_TOKENS // _TILES


def _logits_body(x_ref, gw_ref, gb_ref, out_ref):
    logits = (
        jnp.dot(x_ref[...], gw_ref[...], preferred_element_type=jnp.float32)
        + gb_ref[...]
    )
    out_ref[...] = logits.T


def _route_body(lg_ref, cm_ref, lt_ref, buf_ref, sem):
    wid = lax.axis_index("c") * 16 + lax.axis_index("s")
    base = wid * _TOK_PER_TILE
    pltpu.sync_copy(lg_ref.at[:, pl.ds(base, _TOK_PER_TILE)], lt_ref)
    for g in range(_TOK_PER_TILE // _LANES):
        le = [
            lt_ref[e, pl.ds(g * _LANES, _LANES)] for e in range(NUM_EXPERTS)
        ]
        m1 = le[0]
        for e in range(1, NUM_EXPERTS):
            m1 = jnp.maximum(m1, le[e])
        i1 = jnp.full((_LANES,), NUM_EXPERTS, jnp.int32)
        for e in range(NUM_EXPERTS - 1, -1, -1):
            i1 = jnp.where(le[e] == m1, e, i1)
        neg = jnp.full((_LANES,), -jnp.inf, jnp.float32)
        m2 = jnp.where(i1 == 0, neg, le[0])
        for e in range(1, NUM_EXPERTS):
            m2 = jnp.maximum(m2, jnp.where(i1 == e, neg, le[e]))
        i2 = jnp.full((_LANES,), NUM_EXPERTS, jnp.int32)
        for e in range(NUM_EXPERTS - 1, -1, -1):
            i2 = jnp.where((le[e] == m2) & (i1 != e), e, i2)
        c1 = 1.0 / (1.0 + jnp.exp(m2 - m1))
        c2 = 1.0 - c1
        zero = jnp.zeros((_LANES,), jnp.float32)
        for e in range(NUM_EXPERTS):
            colv = jnp.where(i1 == e, c1, zero) + jnp.where(i2 == e, c2, zero)
            buf_ref[e, pl.ds(g * _LANES, _LANES)] = colv
    pltpu.sync_copy(buf_ref, cm_ref.at[:, pl.ds(base, _TOK_PER_TILE)])


def _moe_body(x_ref, cmt_ref, ew_ref, eb_ref, out_ref):
    e = pl.program_id(1)
    cmt = cmt_ref[...]
    r2 = jax.lax.broadcasted_iota(jnp.int32, (NUM_EXPERTS, 2), 0)
    k2 = jax.lax.broadcasted_iota(jnp.int32, (NUM_EXPERTS, 2), 1)
    sel = (r2 == 2 * e + k2).astype(jnp.float32)
    cab = jax.lax.dot_general(
        cmt, sel, (((0,), (0,)), ((), ())), preferred_element_type=jnp.float32
    )
    ca = cab[:, 0:1]
    cb = cab[:, 1:2]
    ya = jnp.dot(x_ref[...], ew_ref[0], preferred_element_type=jnp.float32)
    yb = jnp.dot(x_ref[...], ew_ref[1], preferred_element_type=jnp.float32)
    y = ca * ya + cb * yb

    @pl.when(e == 0)
    def _init():
        out_ref[...] = y + jax.lax.dot_general(
            cmt,
            eb_ref[...],
            (((0,), (0,)), ((), ())),
            preferred_element_type=jnp.float32,
        )

    @pl.when(e != 0)
    def _acc():
        out_ref[...] = out_ref[...] + y


@jax.jit
def kernel(x, gate_W, gate_b, expert_W, expert_b):
    gb2 = gate_b.reshape(1, NUM_EXPERTS)
    logits_t = pl.pallas_call(
        _logits_body,
        grid=(1,),
        in_specs=[
            pl.BlockSpec((N_TOKENS, D_MODEL), lambda i: (0, 0)),
            pl.BlockSpec((D_MODEL, NUM_EXPERTS), lambda i: (0, 0)),
            pl.BlockSpec((1, NUM_EXPERTS), lambda i: (0, 0)),
        ],
        out_specs=pl.BlockSpec((NUM_EXPERTS, N_TOKENS), lambda i: (0, 0)),
        out_shape=jax.ShapeDtypeStruct((NUM_EXPERTS, N_TOKENS), jnp.float32),
    )(x, gate_W, gb2)

    mesh = plsc.VectorSubcoreMesh(core_axis_name="c", subcore_axis_name="s", num_cores=1)
    route = functools.partial(
        pl.kernel,
        out_type=jax.ShapeDtypeStruct((NUM_EXPERTS, N_TOKENS), jnp.float32),
        mesh=mesh,
        scratch_types=[
            pltpu.VMEM((NUM_EXPERTS, _TOK_PER_TILE), jnp.float32),
            pltpu.VMEM((NUM_EXPERTS, _TOK_PER_TILE), jnp.float32),
            pltpu.SemaphoreType.DMA,
        ],
    )(_route_body)
    cmat_t = route(logits_t)

    n_tb = N_TOKENS // TOKEN_BLOCK
    return pl.pallas_call(
        _moe_body,
        grid=(n_tb, NUM_EXPERTS // 2),
        in_specs=[
            pl.BlockSpec((TOKEN_BLOCK, D_MODEL), lambda t, e: (t, 0)),
            pl.BlockSpec((NUM_EXPERTS, TOKEN_BLOCK), lambda t, e: (0, t)),
            pl.BlockSpec((2, D_MODEL, D_MODEL), lambda t, e: (e, 0, 0)),
            pl.BlockSpec((NUM_EXPERTS, D_MODEL), lambda t, e: (0, 0)),
        ],
        out_specs=pl.BlockSpec((TOKEN_BLOCK, D_MODEL), lambda t, e: (t, 0)),
        out_shape=jax.ShapeDtypeStruct((N_TOKENS, D_MODEL), jnp.float32),
    )(x, cmat_t, expert_W, expert_b)
